# Initial kernel scaffold; baseline (speedup 1.0000x reference)
import jax
import jax.numpy as jnp
from jax.experimental import pallas as pl


def kernel(output, target):
    raise NotImplementedError("write your pallas kernel here")


# speedup vs baseline: 1.0475x; 1.0475x over previous
import jax
import jax.numpy as jnp
from jax.experimental import pallas as pl
from jax.experimental.pallas import tpu as pltpu

_ROWS = 4096
_COLS = 512
_BLOCK_ROWS = 512
_N_BLOCKS = _ROWS // _BLOCK_ROWS
_N_TOTAL = float(_ROWS * _COLS)


def _body(x_ref, y_ref, loss_ref, acc_ref):
    i = pl.program_id(0)

    @pl.when(i == 0)
    def _init():
        acc_ref[0] = 0.0
        acc_ref[1] = 0.0
        acc_ref[2] = 0.0

    x = x_ref[...]
    y = y_ref[...]
    per = jnp.maximum(x, 0.0) - x * y + jnp.log1p(jnp.exp(-jnp.abs(x)))
    acc_ref[0] += jnp.sum(per)
    acc_ref[1] += jnp.sum(per * y)
    acc_ref[2] += jnp.sum(y)

    @pl.when(i == _N_BLOCKS - 1)
    def _fin():
        tot = acc_ref[0]
        fg = acc_ref[1]
        cnt = acc_ref[2]
        bg_cnt = jnp.maximum(_N_TOTAL - cnt, 1.0)
        fg_cnt = jnp.maximum(cnt, 1.0)
        loss_ref[0, 0] = (tot - fg) / bg_cnt + fg / fg_cnt


def kernel(output, target):
    x = output.reshape(_ROWS, _COLS)
    y = target.reshape(_ROWS, _COLS)
    loss = pl.pallas_call(
        _body,
        grid=(_N_BLOCKS,),
        in_specs=[
            pl.BlockSpec((_BLOCK_ROWS, _COLS), lambda i: (i, 0)),
            pl.BlockSpec((_BLOCK_ROWS, _COLS), lambda i: (i, 0)),
        ],
        out_specs=pl.BlockSpec(memory_space=pltpu.SMEM),
        out_shape=jax.ShapeDtypeStruct((1, 1), jnp.float32),
        scratch_shapes=[pltpu.SMEM((3,), jnp.float32)],
    )(x, y)
    return loss[0, 0]
